# Initial kernel scaffold; baseline (speedup 1.0000x reference)
import jax
import jax.numpy as jnp
from jax.experimental import pallas as pl


def kernel(node_attrs, node_feats, edge_attrs, edge_feats, edge_index, W_up, mlp_w0, mlp_w1, mlp_w2, mlp_w3, W_lin, W_skip, update_coeff):
    raise NotImplementedError("write your pallas kernel here")


# speedup vs baseline: 2.0046x; 2.0046x over previous
import functools
import math

import jax
import jax.numpy as jnp
from jax import lax
from jax.experimental import pallas as pl
from jax.experimental.pallas import tpu as pltpu
from jax.experimental.pallas import tpu_sc as plsc

N = 10000
E = 320000
D = 128
DA = 16

NC = 2
NS = 16
NW = NC * NS
EPW = E // NW
CH = 80
CHUNKS = EPW // CH
N_PAD = 10240
RPT = N_PAD // NS

NB_NODE = 10
BN = N // NB_NODE
NB_EDGE = 320
BE = E // NB_EDGE


def _xup_body(nf_ref, w_ref, o_ref):
    o_ref[...] = jnp.dot(nf_ref[...], w_ref[...],
                         preferred_element_type=jnp.float32) * (1.0 / math.sqrt(D))


def _edge_mlp_body(ef_ref, ea_ref, w0_ref, w1_ref, w2_ref, w3_ref, o_ref):
    h = jnp.dot(ef_ref[...], w0_ref[...], preferred_element_type=jnp.float32)
    h = h * (1.0 / math.sqrt(8.0))
    h = h * jax.nn.sigmoid(h)
    h = jnp.dot(h, w1_ref[...], preferred_element_type=jnp.float32) * (1.0 / math.sqrt(64.0))
    h = h * jax.nn.sigmoid(h)
    h = jnp.dot(h, w2_ref[...], preferred_element_type=jnp.float32) * (1.0 / math.sqrt(64.0))
    h = h * jax.nn.sigmoid(h)
    t = jnp.dot(h, w3_ref[...], preferred_element_type=jnp.float32) * (1.0 / math.sqrt(64.0))
    o_ref[...] = t * ea_ref[...]


def _combine_body(p_ref, na_ref, wl_ref, ws_ref, c_ref, o_ref):
    msg = p_ref[0] + p_ref[1]
    msg = jnp.dot(msg, wl_ref[...], preferred_element_type=jnp.float32)
    msg = msg * (1.0 / (math.sqrt(D) * 32.0))
    acc = jnp.zeros((BN, D), jnp.float32)
    for v in range(DA):
        acc = acc + jnp.dot(msg, ws_ref[v], preferred_element_type=jnp.float32) * na_ref[:, v:v + 1]
    sc = acc * (1.0 / math.sqrt(float(D * DA)))
    o_ref[...] = c_ref[0] * msg + c_ref[1] * sc


def _sc_scatter_body(x_hbm, w_hbm, send_hbm, recv_hbm, out_hbm,
                     sidx, ridx, wv, xg, zb, acc, sem1, sem2):
    c = lax.axis_index("c")
    s = lax.axis_index("s")
    wid = c * NS + s

    def _zrow(i, carry):
        for k in range(D // 16):
            zb[i, pl.ds(16 * k, 16)] = jnp.zeros((16,), jnp.float32)
        return carry
    lax.fori_loop(0, 64, _zrow, 0)
    for t in range(RPT // 64):
        pltpu.sync_copy(zb, acc.at[pl.ds(s * RPT + t * 64, 64)])
    plsc.subcore_barrier()

    base_w = wid * EPW

    def _chunk(j, carry):
        base = base_w + j * CH
        pltpu.sync_copy(send_hbm.at[pl.ds(base, CH)], sidx)
        pltpu.sync_copy(recv_hbm.at[pl.ds(base, CH)], ridx)
        cp_w = pltpu.async_copy(w_hbm.at[pl.ds(base, CH)], wv, sem1)
        cp_x = pltpu.async_copy(x_hbm.at[sidx], xg, sem2)
        cp_w.wait()
        cp_x.wait()

        def _mrow(r, carry2):
            for k in range(D // 16):
                sl = pl.ds(16 * k, 16)
                xg[r, sl] = xg[r, sl] * wv[r, sl]
            return carry2
        lax.fori_loop(0, CH, _mrow, 0)
        pltpu.sync_copy(xg, acc.at[ridx], add=True)
        return carry
    lax.fori_loop(0, CHUNKS, _chunk, 0)

    plsc.subcore_barrier()
    pltpu.sync_copy(acc.at[pl.ds(s * RPT, RPT)], out_hbm.at[c, pl.ds(s * RPT, RPT)])


def _make_sc_scatter():
    return pl.kernel(
        _sc_scatter_body,
        out_type=jax.ShapeDtypeStruct((NC, N_PAD, D), jnp.float32),
        mesh=plsc.VectorSubcoreMesh(core_axis_name="c", subcore_axis_name="s",
                                    num_cores=NC, num_subcores=NS),
        scratch_types=[
            pltpu.VMEM((CH,), jnp.int32),
            pltpu.VMEM((CH,), jnp.int32),
            pltpu.VMEM((CH, D), jnp.float32),
            pltpu.VMEM((CH, D), jnp.float32),
            pltpu.VMEM((64, D), jnp.float32),
            pltpu.VMEM_SHARED((N_PAD, D), jnp.float32),
            pltpu.SemaphoreType.DMA,
            pltpu.SemaphoreType.DMA,
        ],
    )


def kernel(node_attrs, node_feats, edge_attrs, edge_feats, edge_index,
           W_up, mlp_w0, mlp_w1, mlp_w2, mlp_w3, W_lin, W_skip, update_coeff):
    sender = edge_index[0].astype(jnp.int32)
    receiver = edge_index[1].astype(jnp.int32)

    x = pl.pallas_call(
        _xup_body,
        grid=(NB_NODE,),
        in_specs=[pl.BlockSpec((BN, D), lambda i: (i, 0)),
                  pl.BlockSpec((D, D), lambda i: (0, 0))],
        out_specs=pl.BlockSpec((BN, D), lambda i: (i, 0)),
        out_shape=jax.ShapeDtypeStruct((N, D), jnp.float32),
    )(node_feats, W_up)

    w_edges = pl.pallas_call(
        _edge_mlp_body,
        grid=(NB_EDGE,),
        in_specs=[pl.BlockSpec((BE, 8), lambda i: (i, 0)),
                  pl.BlockSpec((BE, 1), lambda i: (i, 0)),
                  pl.BlockSpec((8, 64), lambda i: (0, 0)),
                  pl.BlockSpec((64, 64), lambda i: (0, 0)),
                  pl.BlockSpec((64, 64), lambda i: (0, 0)),
                  pl.BlockSpec((64, D), lambda i: (0, 0))],
        out_specs=pl.BlockSpec((BE, D), lambda i: (i, 0)),
        out_shape=jax.ShapeDtypeStruct((E, D), jnp.float32),
    )(edge_feats, edge_attrs, mlp_w0, mlp_w1, mlp_w2, mlp_w3)

    partials = _make_sc_scatter()(x, w_edges, sender, receiver)

    uc = jax.nn.sigmoid(update_coeff[0])
    c_old = lax.rsqrt(uc * uc + 1.0)
    c_new = uc * c_old
    cvec = jnp.stack([c_old, c_new])
    W_skip_t = jnp.transpose(W_skip, (1, 0, 2))

    out = pl.pallas_call(
        _combine_body,
        grid=(NB_NODE,),
        in_specs=[pl.BlockSpec((NC, BN, D), lambda i: (0, i, 0)),
                  pl.BlockSpec((BN, DA), lambda i: (i, 0)),
                  pl.BlockSpec((D, D), lambda i: (0, 0)),
                  pl.BlockSpec((DA, D, D), lambda i: (0, 0, 0)),
                  pl.BlockSpec(memory_space=pltpu.SMEM)],
        out_specs=pl.BlockSpec((BN, D), lambda i: (i, 0)),
        out_shape=jax.ShapeDtypeStruct((N, D), jnp.float32),
    )(partials, node_attrs, W_lin, W_skip_t, cvec)

    return out.reshape(N, D, 1)


# speedup vs baseline: 2.5479x; 1.2710x over previous
import functools
import math

import jax
import jax.numpy as jnp
from jax import lax
from jax.experimental import pallas as pl
from jax.experimental.pallas import tpu as pltpu
from jax.experimental.pallas import tpu_sc as plsc

N = 10000
E = 320000
D = 128
DA = 16

NC = 2
NS = 16
NW = NC * NS
EPW = E // NW
CH = 40
CHUNKS = EPW // CH
NBUF = 4
NIB = 8
ROUNDS = CHUNKS // NIB
N_PAD = 10240
RPT = N_PAD // NS

NB_NODE = 10
BN = N // NB_NODE
NB_EDGE = 320
BE = E // NB_EDGE


def _xup_body(nf_ref, w_ref, o_ref):
    o_ref[...] = jnp.dot(nf_ref[...], w_ref[...],
                         preferred_element_type=jnp.float32) * (1.0 / math.sqrt(D))


def _edge_mlp_body(ef_ref, ea_ref, w0_ref, w1_ref, w2_ref, w3_ref, o_ref):
    h = jnp.dot(ef_ref[...], w0_ref[...], preferred_element_type=jnp.float32)
    h = h * (1.0 / math.sqrt(8.0))
    h = h * jax.nn.sigmoid(h)
    h = jnp.dot(h, w1_ref[...], preferred_element_type=jnp.float32) * (1.0 / math.sqrt(64.0))
    h = h * jax.nn.sigmoid(h)
    h = jnp.dot(h, w2_ref[...], preferred_element_type=jnp.float32) * (1.0 / math.sqrt(64.0))
    h = h * jax.nn.sigmoid(h)
    t = jnp.dot(h, w3_ref[...], preferred_element_type=jnp.float32) * (1.0 / math.sqrt(64.0))
    o_ref[...] = t * ea_ref[...]


def _combine_body(p_ref, na_ref, wl_ref, ws_ref, c_ref, o_ref):
    msg = p_ref[0] + p_ref[1]
    msg = jnp.dot(msg, wl_ref[...], preferred_element_type=jnp.float32)
    msg = msg * (1.0 / (math.sqrt(D) * 32.0))
    acc = jnp.zeros((BN, D), jnp.float32)
    for v in range(DA):
        acc = acc + jnp.dot(msg, ws_ref[v], preferred_element_type=jnp.float32) * na_ref[:, v:v + 1]
    sc = acc * (1.0 / math.sqrt(float(D * DA)))
    o_ref[...] = c_ref[0] * msg + c_ref[1] * sc


def _sc_scatter_body(x_hbm, w_hbm, send_hbm, recv_hbm, out_hbm,
                     sidxb, ridxb, wv, xg, zb, acc, semw, semx, semi):
    c = lax.axis_index("c")
    s = lax.axis_index("s")
    wid = c * NS + s
    base_e = wid * EPW

    def _zrow(i, carry):
        for k in range(D // 16):
            zb[i, pl.ds(16 * k, 16)] = jnp.zeros((16,), jnp.float32)
        return carry
    lax.fori_loop(0, 16, _zrow, 0)
    for t in range(RPT // 16):
        pltpu.sync_copy(zb, acc.at[pl.ds(s * RPT + t * 16, 16)])
    plsc.subcore_barrier()

    def _issue_idx(j, ib):
        pltpu.async_copy(send_hbm.at[pl.ds(base_e + j * CH, CH)], sidxb.at[ib],
                         semi.at[ib])
        pltpu.async_copy(recv_hbm.at[pl.ds(base_e + j * CH, CH)], ridxb.at[ib],
                         semi.at[ib])

    def _wait_idx(ib):
        pltpu.make_async_copy(send_hbm.at[pl.ds(0, CH)], sidxb.at[ib],
                              semi.at[ib]).wait()
        pltpu.make_async_copy(send_hbm.at[pl.ds(0, CH)], ridxb.at[ib],
                              semi.at[ib]).wait()

    def _issue_in(j, b, ib):
        pltpu.async_copy(w_hbm.at[pl.ds(base_e + j * CH, CH)], wv.at[b], semw.at[b])
        pltpu.async_copy(x_hbm.at[sidxb.at[ib]], xg.at[b], semx.at[b])

    def _wait_in(b, ib):
        pltpu.make_async_copy(w_hbm.at[pl.ds(0, CH)], wv.at[b], semw.at[b]).wait()
        pltpu.make_async_copy(x_hbm.at[sidxb.at[ib]], xg.at[b], semx.at[b]).wait()

    def _mult(b):
        def _mrow(r, carry2):
            for k in range(D // 16):
                sl = pl.ds(16 * k, 16)
                xg[b, r, sl] = xg[b, r, sl] * wv[b, r, sl]
            return carry2
        lax.fori_loop(0, CH, _mrow, 0)

    def _scatter(b, ib):
        pltpu.sync_copy(xg.at[b], acc.at[ridxb.at[ib]], add=True)

    for jj in range(NBUF):
        _issue_idx(jj, jj)
    _wait_idx(0)
    _issue_in(0, 0, 0)
    _wait_idx(1)
    _issue_in(1, 1, 1)

    def _slot(j, u):
        ub = u % NBUF
        _wait_in(ub, u)
        _mult(ub)
        _scatter(ub, u)
        b2 = (ub + 2) % NBUF
        i2 = (u + 2) % NIB
        i4 = (u + 4) % NIB

        @pl.when(j + 2 < CHUNKS)
        def _():
            _wait_idx(i2)
            _issue_in(j + 2, b2, i2)

        @pl.when(j + 4 < CHUNKS)
        def _():
            _issue_idx(j + 4, i4)

    def _round(t, carry):
        j0 = NIB * t
        for u in range(NIB):
            _slot(j0 + u, u)
        return carry
    lax.fori_loop(0, ROUNDS, _round, 0)

    for j in range(NIB * ROUNDS, CHUNKS):
        u = j % NIB
        ub = u % NBUF
        _wait_in(ub, u)
        _mult(ub)
        _scatter(ub, u)

    plsc.subcore_barrier()
    pltpu.sync_copy(acc.at[pl.ds(s * RPT, RPT)], out_hbm.at[c, pl.ds(s * RPT, RPT)])


def _make_sc_scatter():
    return pl.kernel(
        _sc_scatter_body,
        out_type=jax.ShapeDtypeStruct((NC, N_PAD, D), jnp.float32),
        mesh=plsc.VectorSubcoreMesh(core_axis_name="c", subcore_axis_name="s",
                                    num_cores=NC, num_subcores=NS),
        scratch_types=[
            pltpu.VMEM((NIB, CH), jnp.int32),
            pltpu.VMEM((NIB, CH), jnp.int32),
            pltpu.VMEM((NBUF, CH, D), jnp.float32),
            pltpu.VMEM((NBUF, CH, D), jnp.float32),
            pltpu.VMEM((16, D), jnp.float32),
            pltpu.VMEM_SHARED((N_PAD, D), jnp.float32),
            pltpu.SemaphoreType.DMA((NBUF,)),
            pltpu.SemaphoreType.DMA((NBUF,)),
            pltpu.SemaphoreType.DMA((NIB,)),
        ],
    )


def kernel(node_attrs, node_feats, edge_attrs, edge_feats, edge_index,
           W_up, mlp_w0, mlp_w1, mlp_w2, mlp_w3, W_lin, W_skip, update_coeff):
    sender = edge_index[0].astype(jnp.int32)
    receiver = edge_index[1].astype(jnp.int32)

    x = pl.pallas_call(
        _xup_body,
        grid=(NB_NODE,),
        in_specs=[pl.BlockSpec((BN, D), lambda i: (i, 0)),
                  pl.BlockSpec((D, D), lambda i: (0, 0))],
        out_specs=pl.BlockSpec((BN, D), lambda i: (i, 0)),
        out_shape=jax.ShapeDtypeStruct((N, D), jnp.float32),
    )(node_feats, W_up)

    w_edges = pl.pallas_call(
        _edge_mlp_body,
        grid=(NB_EDGE,),
        in_specs=[pl.BlockSpec((BE, 8), lambda i: (i, 0)),
                  pl.BlockSpec((BE, 1), lambda i: (i, 0)),
                  pl.BlockSpec((8, 64), lambda i: (0, 0)),
                  pl.BlockSpec((64, 64), lambda i: (0, 0)),
                  pl.BlockSpec((64, 64), lambda i: (0, 0)),
                  pl.BlockSpec((64, D), lambda i: (0, 0))],
        out_specs=pl.BlockSpec((BE, D), lambda i: (i, 0)),
        out_shape=jax.ShapeDtypeStruct((E, D), jnp.float32),
    )(edge_feats, edge_attrs, mlp_w0, mlp_w1, mlp_w2, mlp_w3)

    partials = _make_sc_scatter()(x, w_edges, sender, receiver)

    uc = jax.nn.sigmoid(update_coeff[0])
    c_old = lax.rsqrt(uc * uc + 1.0)
    c_new = uc * c_old
    cvec = jnp.stack([c_old, c_new])
    W_skip_t = jnp.transpose(W_skip, (1, 0, 2))

    out = pl.pallas_call(
        _combine_body,
        grid=(NB_NODE,),
        in_specs=[pl.BlockSpec((NC, BN, D), lambda i: (0, i, 0)),
                  pl.BlockSpec((BN, DA), lambda i: (i, 0)),
                  pl.BlockSpec((D, D), lambda i: (0, 0)),
                  pl.BlockSpec((DA, D, D), lambda i: (0, 0, 0)),
                  pl.BlockSpec(memory_space=pltpu.SMEM)],
        out_specs=pl.BlockSpec((BN, D), lambda i: (i, 0)),
        out_shape=jax.ShapeDtypeStruct((N, D), jnp.float32),
    )(partials, node_attrs, W_lin, W_skip_t, cvec)

    return out.reshape(N, D, 1)


# speedup vs baseline: 2.8859x; 1.1326x over previous
import functools
import math

import jax
import jax.numpy as jnp
from jax import lax
from jax.experimental import pallas as pl
from jax.experimental.pallas import tpu as pltpu
from jax.experimental.pallas import tpu_sc as plsc

N = 10000
E = 320000
D = 128
DA = 16

NC = 2
NS = 16
NW = NC * NS
EPW = E // NW
CH = 40
CHUNKS = EPW // CH
NBUF = 4
NIB = 8
ROUNDS = CHUNKS // NIB
N_PAD = 10240
RPT = N_PAD // NS

NB_NODE = 10
BN = N // NB_NODE
NB_EDGE = 160
BE = E // NB_EDGE


def _xup_body(nf_ref, w_ref, o_ref):
    o_ref[...] = jnp.dot(nf_ref[...], w_ref[...],
                         preferred_element_type=jnp.float32) * (1.0 / math.sqrt(D))


def _edge_mlp_body(ef_ref, ea_ref, w0_ref, w1_ref, w2_ref, w3_ref, o_ref):
    def dot16(a, b):
        return jnp.dot(a.astype(jnp.bfloat16), b.astype(jnp.bfloat16),
                       preferred_element_type=jnp.float32)
    h = dot16(ef_ref[...], w0_ref[...]) * (1.0 / math.sqrt(8.0))
    h = h * jax.nn.sigmoid(h)
    h = dot16(h, w1_ref[...]) * (1.0 / math.sqrt(64.0))
    h = h * jax.nn.sigmoid(h)
    h = dot16(h, w2_ref[...]) * (1.0 / math.sqrt(64.0))
    h = h * jax.nn.sigmoid(h)
    h = h * ea_ref[...]
    o_ref[...] = dot16(h, w3_ref[...]) * (1.0 / math.sqrt(64.0))


def _combine_body(p_ref, na_ref, wl_ref, ws_ref, c_ref, o_ref):
    msg = p_ref[0] + p_ref[1]
    msg = jnp.dot(msg, wl_ref[...], preferred_element_type=jnp.float32)
    msg = msg * (1.0 / (math.sqrt(D) * 32.0))
    acc = jnp.zeros((BN, D), jnp.float32)
    for v in range(DA):
        acc = acc + jnp.dot(msg, ws_ref[v], preferred_element_type=jnp.float32) * na_ref[:, v:v + 1]
    sc = acc * (1.0 / math.sqrt(float(D * DA)))
    o_ref[...] = c_ref[0] * msg + c_ref[1] * sc


def _sc_scatter_body(x_hbm, w_hbm, send_hbm, recv_hbm, out_hbm,
                     sidxb, ridxb, wv, xg, zb, acc, semw, semx, semi):
    c = lax.axis_index("c")
    s = lax.axis_index("s")
    wid = c * NS + s
    base_e = wid * EPW

    def _zrow(i, carry):
        for k in range(D // 16):
            zb[i, pl.ds(16 * k, 16)] = jnp.zeros((16,), jnp.float32)
        return carry
    lax.fori_loop(0, 16, _zrow, 0)
    for t in range(RPT // 16):
        pltpu.sync_copy(zb, acc.at[pl.ds(s * RPT + t * 16, 16)])
    plsc.subcore_barrier()

    def _issue_idx(j, ib):
        pltpu.async_copy(send_hbm.at[pl.ds(base_e + j * CH, CH)], sidxb.at[ib],
                         semi.at[ib])
        pltpu.async_copy(recv_hbm.at[pl.ds(base_e + j * CH, CH)], ridxb.at[ib],
                         semi.at[ib])

    def _wait_idx(ib):
        pltpu.make_async_copy(send_hbm.at[pl.ds(0, CH)], sidxb.at[ib],
                              semi.at[ib]).wait()
        pltpu.make_async_copy(send_hbm.at[pl.ds(0, CH)], ridxb.at[ib],
                              semi.at[ib]).wait()

    def _issue_in(j, b, ib):
        pltpu.async_copy(w_hbm.at[pl.ds(base_e + j * CH, CH)], wv.at[b], semw.at[b])
        pltpu.async_copy(x_hbm.at[sidxb.at[ib]], xg.at[b], semx.at[b])

    def _wait_in(b, ib):
        pltpu.make_async_copy(w_hbm.at[pl.ds(0, CH)], wv.at[b], semw.at[b]).wait()
        pltpu.make_async_copy(x_hbm.at[sidxb.at[ib]], xg.at[b], semx.at[b]).wait()

    def _mult(b):
        def _mrow(r, carry2):
            for k in range(D // 16):
                sl = pl.ds(16 * k, 16)
                xg[b, r, sl] = xg[b, r, sl] * wv[b, r, sl]
            return carry2
        lax.fori_loop(0, CH, _mrow, 0)

    def _scatter(b, ib):
        pltpu.sync_copy(xg.at[b], acc.at[ridxb.at[ib]], add=True)

    for jj in range(NBUF):
        _issue_idx(jj, jj)
    _wait_idx(0)
    _issue_in(0, 0, 0)
    _wait_idx(1)
    _issue_in(1, 1, 1)

    def _slot(j, u):
        ub = u % NBUF
        _wait_in(ub, u)
        _mult(ub)
        _scatter(ub, u)
        b2 = (ub + 2) % NBUF
        i2 = (u + 2) % NIB
        i4 = (u + 4) % NIB

        @pl.when(j + 2 < CHUNKS)
        def _():
            _wait_idx(i2)
            _issue_in(j + 2, b2, i2)

        @pl.when(j + 4 < CHUNKS)
        def _():
            _issue_idx(j + 4, i4)

    def _round(t, carry):
        j0 = NIB * t
        for u in range(NIB):
            _slot(j0 + u, u)
        return carry
    lax.fori_loop(0, ROUNDS, _round, 0)

    for j in range(NIB * ROUNDS, CHUNKS):
        u = j % NIB
        ub = u % NBUF
        _wait_in(ub, u)
        _mult(ub)
        _scatter(ub, u)

    plsc.subcore_barrier()
    pltpu.sync_copy(acc.at[pl.ds(s * RPT, RPT)], out_hbm.at[c, pl.ds(s * RPT, RPT)])


def _make_sc_scatter():
    return pl.kernel(
        _sc_scatter_body,
        out_type=jax.ShapeDtypeStruct((NC, N_PAD, D), jnp.float32),
        mesh=plsc.VectorSubcoreMesh(core_axis_name="c", subcore_axis_name="s",
                                    num_cores=NC, num_subcores=NS),
        scratch_types=[
            pltpu.VMEM((NIB, CH), jnp.int32),
            pltpu.VMEM((NIB, CH), jnp.int32),
            pltpu.VMEM((NBUF, CH, D), jnp.float32),
            pltpu.VMEM((NBUF, CH, D), jnp.float32),
            pltpu.VMEM((16, D), jnp.float32),
            pltpu.VMEM_SHARED((N_PAD, D), jnp.float32),
            pltpu.SemaphoreType.DMA((NBUF,)),
            pltpu.SemaphoreType.DMA((NBUF,)),
            pltpu.SemaphoreType.DMA((NIB,)),
        ],
    )


def kernel(node_attrs, node_feats, edge_attrs, edge_feats, edge_index,
           W_up, mlp_w0, mlp_w1, mlp_w2, mlp_w3, W_lin, W_skip, update_coeff):
    sender = edge_index[0].astype(jnp.int32)
    receiver = edge_index[1].astype(jnp.int32)

    x = pl.pallas_call(
        _xup_body,
        grid=(NB_NODE,),
        in_specs=[pl.BlockSpec((BN, D), lambda i: (i, 0)),
                  pl.BlockSpec((D, D), lambda i: (0, 0))],
        out_specs=pl.BlockSpec((BN, D), lambda i: (i, 0)),
        out_shape=jax.ShapeDtypeStruct((N, D), jnp.float32),
    )(node_feats, W_up)

    w_edges = pl.pallas_call(
        _edge_mlp_body,
        grid=(NB_EDGE,),
        in_specs=[pl.BlockSpec((BE, 8), lambda i: (i, 0)),
                  pl.BlockSpec((BE, 1), lambda i: (i, 0)),
                  pl.BlockSpec((8, 64), lambda i: (0, 0)),
                  pl.BlockSpec((64, 64), lambda i: (0, 0)),
                  pl.BlockSpec((64, 64), lambda i: (0, 0)),
                  pl.BlockSpec((64, D), lambda i: (0, 0))],
        out_specs=pl.BlockSpec((BE, D), lambda i: (i, 0)),
        out_shape=jax.ShapeDtypeStruct((E, D), jnp.float32),
    )(edge_feats, edge_attrs, mlp_w0, mlp_w1, mlp_w2, mlp_w3)

    partials = _make_sc_scatter()(x, w_edges, sender, receiver)

    uc = jax.nn.sigmoid(update_coeff[0])
    c_old = lax.rsqrt(uc * uc + 1.0)
    c_new = uc * c_old
    cvec = jnp.stack([c_old, c_new])
    W_skip_t = jnp.transpose(W_skip, (1, 0, 2))

    out = pl.pallas_call(
        _combine_body,
        grid=(NB_NODE,),
        in_specs=[pl.BlockSpec((NC, BN, D), lambda i: (0, i, 0)),
                  pl.BlockSpec((BN, DA), lambda i: (i, 0)),
                  pl.BlockSpec((D, D), lambda i: (0, 0)),
                  pl.BlockSpec((DA, D, D), lambda i: (0, 0, 0)),
                  pl.BlockSpec(memory_space=pltpu.SMEM)],
        out_specs=pl.BlockSpec((BN, D), lambda i: (i, 0)),
        out_shape=jax.ShapeDtypeStruct((N, D), jnp.float32),
    )(partials, node_attrs, W_lin, W_skip_t, cvec)

    return out.reshape(N, D, 1)


# speedup vs baseline: 3.1110x; 1.0780x over previous
import functools
import math

import jax
import jax.numpy as jnp
from jax import lax
from jax.experimental import pallas as pl
from jax.experimental.pallas import tpu as pltpu
from jax.experimental.pallas import tpu_sc as plsc

N = 10000
E = 320000
D = 128
DA = 16

NC = 2
NS = 16
NW = NC * NS
EPW = E // NW
CH = 40
CHUNKS = EPW // CH
NBUF = 4
NIB = 8
ROUNDS = CHUNKS // NIB
N_PAD = 10240
RPT = N_PAD // NS

NB_NODE = 10
BN = N // NB_NODE
NSPLIT = 2
NB_EDGE = 160
BE = E // NB_EDGE


def _xup_body(nf_ref, w_ref, o_ref):
    o_ref[...] = jnp.dot(nf_ref[...], w_ref[...],
                         preferred_element_type=jnp.float32) * (1.0 / math.sqrt(D))


def _edge_mlp_body(ef_ref, ea_ref, w0_ref, w1_ref, w2_ref, w3_ref, o_ref):
    def dot16(a, b):
        return jnp.dot(a.astype(jnp.bfloat16), b.astype(jnp.bfloat16),
                       preferred_element_type=jnp.float32)
    h = dot16(ef_ref[...], w0_ref[...]) * (1.0 / math.sqrt(8.0))
    h = h * jax.nn.sigmoid(h)
    h = dot16(h, w1_ref[...]) * (1.0 / math.sqrt(64.0))
    h = h * jax.nn.sigmoid(h)
    h = dot16(h, w2_ref[...]) * (1.0 / math.sqrt(64.0))
    h = h * jax.nn.sigmoid(h)
    h = h * ea_ref[...]
    o_ref[...] = dot16(h, w3_ref[...]) * (1.0 / math.sqrt(64.0))


def _combine_body(p_ref, na_ref, wl_ref, ws_ref, c_ref, o_ref):
    msg = p_ref[0] + p_ref[1]
    msg = jnp.dot(msg, wl_ref[...], preferred_element_type=jnp.float32)
    msg = msg * (1.0 / (math.sqrt(D) * 32.0))
    acc = jnp.zeros((BN, D), jnp.float32)
    for v in range(DA):
        acc = acc + jnp.dot(msg, ws_ref[v], preferred_element_type=jnp.float32) * na_ref[:, v:v + 1]
    sc = acc * (1.0 / math.sqrt(float(D * DA)))
    o_ref[...] = c_ref[0] * msg + c_ref[1] * sc


def _make_sc_body(epw, seeded):
    chunks = epw // CH
    rounds = chunks // NIB

    def _body(x_hbm, w_hbm, send_hbm, recv_hbm, prev_hbm, out_hbm,
              sidxb, ridxb, wv, xg, zb, acc, semw, semx, semi):
        c = lax.axis_index("c")
        s = lax.axis_index("s")
        wid = c * NS + s
        base_e = wid * epw

        if seeded:
            pltpu.sync_copy(prev_hbm.at[c, pl.ds(s * RPT, RPT)],
                            acc.at[pl.ds(s * RPT, RPT)])
        else:
            def _zrow(i, carry):
                for k in range(D // 16):
                    zb[i, pl.ds(16 * k, 16)] = jnp.zeros((16,), jnp.float32)
                return carry
            lax.fori_loop(0, 16, _zrow, 0)
            for t in range(RPT // 16):
                pltpu.sync_copy(zb, acc.at[pl.ds(s * RPT + t * 16, 16)])
        plsc.subcore_barrier()

        def _issue_idx(j, ib):
            pltpu.async_copy(send_hbm.at[pl.ds(base_e + j * CH, CH)], sidxb.at[ib],
                             semi.at[ib])
            pltpu.async_copy(recv_hbm.at[pl.ds(base_e + j * CH, CH)], ridxb.at[ib],
                             semi.at[ib])

        def _wait_idx(ib):
            pltpu.make_async_copy(send_hbm.at[pl.ds(0, CH)], sidxb.at[ib],
                                  semi.at[ib]).wait()
            pltpu.make_async_copy(send_hbm.at[pl.ds(0, CH)], ridxb.at[ib],
                                  semi.at[ib]).wait()

        def _issue_in(j, b, ib):
            pltpu.async_copy(w_hbm.at[pl.ds(base_e + j * CH, CH)], wv.at[b], semw.at[b])
            pltpu.async_copy(x_hbm.at[sidxb.at[ib]], xg.at[b], semx.at[b])

        def _wait_in(b, ib):
            pltpu.make_async_copy(w_hbm.at[pl.ds(0, CH)], wv.at[b], semw.at[b]).wait()
            pltpu.make_async_copy(x_hbm.at[sidxb.at[ib]], xg.at[b], semx.at[b]).wait()

        def _mult(b):
            def _mrow(r, carry2):
                for k in range(D // 16):
                    sl = pl.ds(16 * k, 16)
                    xg[b, r, sl] = xg[b, r, sl] * wv[b, r, sl]
                return carry2
            lax.fori_loop(0, CH, _mrow, 0)

        def _scatter(b, ib):
            pltpu.sync_copy(xg.at[b], acc.at[ridxb.at[ib]], add=True)

        for jj in range(NBUF):
            _issue_idx(jj, jj)
        _wait_idx(0)
        _issue_in(0, 0, 0)
        _wait_idx(1)
        _issue_in(1, 1, 1)

        def _slot(j, u):
            ub = u % NBUF
            _wait_in(ub, u)
            _mult(ub)
            _scatter(ub, u)
            b2 = (ub + 2) % NBUF
            i2 = (u + 2) % NIB
            i4 = (u + 4) % NIB

            @pl.when(j + 2 < chunks)
            def _():
                _wait_idx(i2)
                _issue_in(j + 2, b2, i2)

            @pl.when(j + 4 < chunks)
            def _():
                _issue_idx(j + 4, i4)

        def _round(t, carry):
            j0 = NIB * t
            for u in range(NIB):
                _slot(j0 + u, u)
            return carry
        lax.fori_loop(0, rounds, _round, 0)

        for j in range(NIB * rounds, chunks):
            u = j % NIB
            ub = u % NBUF
            _wait_in(ub, u)
            _mult(ub)
            _scatter(ub, u)
            if j + 2 < chunks:
                i2 = (u + 2) % NIB
                _wait_idx(i2)
                _issue_in(j + 2, (ub + 2) % NBUF, i2)
            if j + 4 < chunks:
                _issue_idx(j + 4, (u + 4) % NIB)

        plsc.subcore_barrier()
        pltpu.sync_copy(acc.at[pl.ds(s * RPT, RPT)], out_hbm.at[c, pl.ds(s * RPT, RPT)])

    return _body


def _make_sc_scatter(epw, seeded):
    return pl.kernel(
        _make_sc_body(epw, seeded),
        out_type=jax.ShapeDtypeStruct((NC, N_PAD, D), jnp.float32),
        mesh=plsc.VectorSubcoreMesh(core_axis_name="c", subcore_axis_name="s",
                                    num_cores=NC, num_subcores=NS),
        scratch_types=[
            pltpu.VMEM((NIB, CH), jnp.int32),
            pltpu.VMEM((NIB, CH), jnp.int32),
            pltpu.VMEM((NBUF, CH, D), jnp.float32),
            pltpu.VMEM((NBUF, CH, D), jnp.float32),
            pltpu.VMEM((16, D), jnp.float32),
            pltpu.VMEM_SHARED((N_PAD, D), jnp.float32),
            pltpu.SemaphoreType.DMA((NBUF,)),
            pltpu.SemaphoreType.DMA((NBUF,)),
            pltpu.SemaphoreType.DMA((NIB,)),
        ],
    )


def kernel(node_attrs, node_feats, edge_attrs, edge_feats, edge_index,
           W_up, mlp_w0, mlp_w1, mlp_w2, mlp_w3, W_lin, W_skip, update_coeff):
    sender = edge_index[0].astype(jnp.int32)
    receiver = edge_index[1].astype(jnp.int32)
    eh = E // NSPLIT

    x = pl.pallas_call(
        _xup_body,
        grid=(NB_NODE,),
        in_specs=[pl.BlockSpec((BN, D), lambda i: (i, 0)),
                  pl.BlockSpec((D, D), lambda i: (0, 0))],
        out_specs=pl.BlockSpec((BN, D), lambda i: (i, 0)),
        out_shape=jax.ShapeDtypeStruct((N, D), jnp.float32),
    )(node_feats, W_up)

    partials = jnp.zeros((8, D), jnp.float32)
    for k in range(NSPLIT):
        lo = k * eh
        w_k = pl.pallas_call(
            _edge_mlp_body,
            grid=(NB_EDGE // NSPLIT,),
            in_specs=[pl.BlockSpec((BE, 8), lambda i: (i, 0)),
                      pl.BlockSpec((BE, 1), lambda i: (i, 0)),
                      pl.BlockSpec((8, 64), lambda i: (0, 0)),
                      pl.BlockSpec((64, 64), lambda i: (0, 0)),
                      pl.BlockSpec((64, 64), lambda i: (0, 0)),
                      pl.BlockSpec((64, D), lambda i: (0, 0))],
            out_specs=pl.BlockSpec((BE, D), lambda i: (i, 0)),
            out_shape=jax.ShapeDtypeStruct((eh, D), jnp.float32),
        )(lax.dynamic_slice_in_dim(edge_feats, lo, eh, 0),
          lax.dynamic_slice_in_dim(edge_attrs, lo, eh, 0),
          mlp_w0, mlp_w1, mlp_w2, mlp_w3)
        partials = _make_sc_scatter(eh // NW, k > 0)(
            x, w_k,
            lax.dynamic_slice_in_dim(sender, lo, eh, 0),
            lax.dynamic_slice_in_dim(receiver, lo, eh, 0),
            partials)

    uc = jax.nn.sigmoid(update_coeff[0])
    c_old = lax.rsqrt(uc * uc + 1.0)
    c_new = uc * c_old
    cvec = jnp.stack([c_old, c_new])
    W_skip_t = jnp.transpose(W_skip, (1, 0, 2))

    out = pl.pallas_call(
        _combine_body,
        grid=(NB_NODE,),
        in_specs=[pl.BlockSpec((NC, BN, D), lambda i: (0, i, 0)),
                  pl.BlockSpec((BN, DA), lambda i: (i, 0)),
                  pl.BlockSpec((D, D), lambda i: (0, 0)),
                  pl.BlockSpec((DA, D, D), lambda i: (0, 0, 0)),
                  pl.BlockSpec(memory_space=pltpu.SMEM)],
        out_specs=pl.BlockSpec((BN, D), lambda i: (i, 0)),
        out_shape=jax.ShapeDtypeStruct((N, D), jnp.float32),
    )(partials, node_attrs, W_lin, W_skip_t, cvec)

    return out.reshape(N, D, 1)
